# baseline (speedup 1.0000x reference)
import functools

import jax
import jax.numpy as jnp
from jax import lax
from jax.experimental import pallas as pl
from jax.experimental.pallas import tpu as pltpu
from jax.experimental.pallas import tpu_sc as plsc

NC = 2
NS = 16
NW = NC * NS
CHUNK = 8


def _tables_body(x_ref, m_ref, c_ref, b2_ref, t_ref, q_ref):
    y = lax.dot_general(x_ref[...], m_ref[...], (((0,), (0,)), ((), ())),
                        preferred_element_type=jnp.float32)
    y = y + c_ref[...]
    col = lax.broadcasted_iota(jnp.int32, y.shape, 1)
    t_ref[...] = jnp.where(col < 64, jnp.maximum(y, 0.0), y)
    q_ref[...] = y[:, 64:] + b2_ref[...]


def _make_tables(xp, M, cvec, b2vec, NP, BN):
    grid = NP // BN
    return pl.pallas_call(
        _tables_body,
        grid=(grid,),
        in_specs=[
            pl.BlockSpec((128, BN), lambda i: (0, i)),
            pl.BlockSpec((128, 128), lambda i: (0, 0)),
            pl.BlockSpec((1, 128), lambda i: (0, 0)),
            pl.BlockSpec((1, 64), lambda i: (0, 0)),
        ],
        out_specs=[
            pl.BlockSpec((BN, 128), lambda i: (i, 0)),
            pl.BlockSpec((BN, 64), lambda i: (i, 0)),
        ],
        out_shape=[
            jax.ShapeDtypeStruct((NP, 128), jnp.float32),
            jax.ShapeDtypeStruct((NP, 64), jnp.float32),
        ],
    )(xp, M, cvec, b2vec)


def _make_sc_kernel(NP, K):
    NPW = NP // NW
    n_chunks = NPW // CHUNK
    E = CHUNK * K

    mesh = plsc.VectorSubcoreMesh(core_axis_name="c", subcore_axis_name="s")

    @functools.partial(
        pl.kernel,
        out_type=jax.ShapeDtypeStruct((NP, 128), jnp.float32),
        mesh=mesh,
        scratch_types=[
            pltpu.VMEM((n_chunks, E), jnp.int32),
            pltpu.VMEM((n_chunks, E), jnp.int32),
            pltpu.VMEM((2, E, 128), jnp.float32),
            pltpu.VMEM((2, E, 64), jnp.float32),
            pltpu.VMEM((2, CHUNK, 128), jnp.float32),
            pltpu.SemaphoreType.DMA,
            pltpu.SemaphoreType.DMA,
            pltpu.SemaphoreType.DMA,
            pltpu.SemaphoreType.DMA,
            pltpu.SemaphoreType.DMA,
            pltpu.SemaphoreType.DMA,
        ],
        compiler_params=pltpu.CompilerParams(use_tc_tiling_on_sc=False),
    )
    def sc_kernel(t_hbm, q_hbm, ii_hbm, jj_hbm, out_hbm,
                  ii_v, jj_v, tbuf, qbuf, obuf,
                  sem_t0, sem_t1, sem_q0, sem_q1, sem_o0, sem_o1):
        sems_t = (sem_t0, sem_t1)
        sems_q = (sem_q0, sem_q1)
        sems_o = (sem_o0, sem_o1)
        wid = lax.axis_index("c") * NS + lax.axis_index("s")
        pltpu.sync_copy(ii_hbm.at[wid], ii_v)
        pltpu.sync_copy(jj_hbm.at[wid], jj_v)
        node_base = wid * NPW
        inv_k = 1.0 / K

        def issue_gather(chunk, b):
            pltpu.async_copy(t_hbm.at[ii_v.at[chunk]], tbuf.at[b], sems_t[b])
            pltpu.async_copy(q_hbm.at[jj_v.at[chunk]], qbuf.at[b], sems_q[b])

        def wait_gather(chunk, b):
            pltpu.make_async_copy(
                t_hbm.at[ii_v.at[chunk]], tbuf.at[b], sems_t[b]).wait()
            pltpu.make_async_copy(
                q_hbm.at[jj_v.at[chunk]], qbuf.at[b], sems_q[b]).wait()

        def out_dst(chunk):
            return out_hbm.at[pl.ds(node_base + chunk * CHUNK, CHUNK)]

        issue_gather(0, 0)
        issue_gather(1, 1)

        def outer(o, _):
            for b in range(2):
                chunk = o * 2 + b
                wait_gather(chunk, b)

                @pl.when(o > 0)
                def _():
                    pltpu.make_async_copy(
                        obuf.at[b], out_dst(chunk - 2), sems_o[b]).wait()

                def node_body(m, _):
                    accs = [jnp.zeros((16,), jnp.float32) for _ in range(8)]
                    r0 = m * K
                    for k in range(K):
                        r = r0 + k
                        for ci in range(4):
                            accs[ci] = accs[ci] + tbuf[b, r, pl.ds(ci * 16, 16)]
                        for ci in range(4):
                            diff = (qbuf[b, r, pl.ds(ci * 16, 16)]
                                    - tbuf[b, r, pl.ds(64 + ci * 16, 16)])
                            accs[4 + ci] = accs[4 + ci] + jnp.maximum(diff, 0.0)
                    for ci in range(8):
                        obuf[b, m, pl.ds(ci * 16, 16)] = accs[ci] * inv_k
                    return 0

                lax.fori_loop(0, CHUNK, node_body, 0)
                pltpu.async_copy(obuf.at[b], out_dst(chunk), sems_o[b])

                @pl.when(chunk + 2 < n_chunks)
                def _():
                    issue_gather(chunk + 2, b)
            return 0

        lax.fori_loop(0, n_chunks // 2, outer, 0)
        for b in range(2):
            pltpu.make_async_copy(
                obuf.at[b], out_dst(n_chunks - 2 + b), sems_o[b]).wait()

    return sc_kernel


def kernel(x, edge_index, W, b):
    B, C, N, _ = x.shape
    K = edge_index.shape[-1]
    NP = ((N + NW * CHUNK - 1) // (NW * CHUNK)) * (NW * CHUNK)

    M = jnp.zeros((128, 128), jnp.float32)
    M = M.at[0:64, 0:32].set(W[0:32].T).at[64:128, 32:64].set(W[32:64].T)
    M = M.at[0:64, 64:96].set(W[64:96].T).at[64:128, 96:128].set(W[96:128].T)
    b1, b2 = b[:64], b[64:]
    cvec = jnp.concatenate([b1, -b2]).reshape(1, 128)
    b2vec = b2.reshape(1, 64)

    xp = jnp.pad(x[0, :, :, 0], ((0, 0), (0, NP - N)))

    T, Q = _make_tables(xp, M, cvec, b2vec, NP, 512)

    NPW = NP // NW
    n_chunks = NPW // CHUNK
    ii = jnp.pad(edge_index[1, 0], ((0, NP - N), (0, 0)))
    jj = jnp.pad(edge_index[0, 0], ((0, NP - N), (0, 0)))
    ii = ii.reshape(NW, n_chunks, CHUNK * K)
    jj = jj.reshape(NW, n_chunks, CHUNK * K)

    out = _make_sc_kernel(NP, K)(T, Q, ii, jj)

    return out[:N].T[None, :, :, None]


# speedup vs baseline: 1.3469x; 1.3469x over previous
import functools

import jax
import jax.numpy as jnp
import numpy as np
from jax import lax
from jax.experimental import pallas as pl
from jax.experimental.pallas import tpu as pltpu
from jax.experimental.pallas import tpu_sc as plsc

NC = 2
NS = 16
NW = NC * NS
CHUNK = 8

_PERM = np.empty(128, np.int32)
for _j in range(4):
    for _t in range(16):
        _PERM[32 * _j + 2 * _t] = 32 * _j + _t
        _PERM[32 * _j + 2 * _t + 1] = 32 * _j + 16 + _t


def _tables_body(x_ref, m_ref, c_ref, b2_ref, t_ref, q_ref):
    y = lax.dot_general(x_ref[...], m_ref[...], (((0,), (0,)), ((), ())),
                        preferred_element_type=jnp.float32)
    y = y + c_ref[...]
    col = lax.broadcasted_iota(jnp.int32, y.shape, 1)
    t_ref[...] = jnp.where(col < 64, jnp.maximum(y, 0.0), y).astype(jnp.bfloat16)
    q_ref[...] = (y[:, 64:] + b2_ref[...]).astype(jnp.bfloat16)


def _make_tables(xp, M, cvec, b2vec, NP, BN):
    grid = NP // BN
    return pl.pallas_call(
        _tables_body,
        grid=(grid,),
        in_specs=[
            pl.BlockSpec((128, BN), lambda i: (0, i)),
            pl.BlockSpec((128, 128), lambda i: (0, 0)),
            pl.BlockSpec((1, 128), lambda i: (0, 0)),
            pl.BlockSpec((1, 64), lambda i: (0, 0)),
        ],
        out_specs=[
            pl.BlockSpec((BN, 128), lambda i: (i, 0)),
            pl.BlockSpec((BN, 64), lambda i: (i, 0)),
        ],
        out_shape=[
            jax.ShapeDtypeStruct((NP, 128), jnp.bfloat16),
            jax.ShapeDtypeStruct((NP, 64), jnp.bfloat16),
        ],
    )(xp, M, cvec, b2vec)


def _make_sc_kernel(NP, K):
    NPW = NP // NW
    n_chunks = NPW // CHUNK
    E = CHUNK * K

    mesh = plsc.VectorSubcoreMesh(core_axis_name="c", subcore_axis_name="s")

    @functools.partial(
        pl.kernel,
        out_type=jax.ShapeDtypeStruct((NP, 128), jnp.float32),
        mesh=mesh,
        scratch_types=[
            pltpu.VMEM((n_chunks, E), jnp.int32),
            pltpu.VMEM((n_chunks, E), jnp.int32),
            pltpu.VMEM((2, E, 128), jnp.bfloat16),
            pltpu.VMEM((2, E, 64), jnp.bfloat16),
            pltpu.VMEM((2, CHUNK, 128), jnp.float32),
            pltpu.SemaphoreType.DMA,
            pltpu.SemaphoreType.DMA,
            pltpu.SemaphoreType.DMA,
            pltpu.SemaphoreType.DMA,
            pltpu.SemaphoreType.DMA,
            pltpu.SemaphoreType.DMA,
        ],
        compiler_params=pltpu.CompilerParams(
            use_tc_tiling_on_sc=False, needs_layout_passes=False),
    )
    def sc_kernel(t_hbm, q_hbm, ii_hbm, jj_hbm, out_hbm,
                  ii_v, jj_v, tbuf, qbuf, obuf,
                  sem_t0, sem_t1, sem_q0, sem_q1, sem_o0, sem_o1):
        sems_t = (sem_t0, sem_t1)
        sems_q = (sem_q0, sem_q1)
        sems_o = (sem_o0, sem_o1)
        wid = lax.axis_index("s") * NC + (1 - lax.axis_index("c"))
        pltpu.sync_copy(ii_hbm.at[wid], ii_v)
        pltpu.sync_copy(jj_hbm.at[wid], jj_v)
        node_base = wid * NPW
        inv_k = 1.0 / K
        unpack = functools.partial(
            plsc.unpack, format=plsc.PackFormat.INTERLEAVED)

        def issue_gather(chunk, b):
            pltpu.async_copy(t_hbm.at[ii_v.at[chunk]], tbuf.at[b], sems_t[b])
            pltpu.async_copy(q_hbm.at[jj_v.at[chunk]], qbuf.at[b], sems_q[b])

        def wait_gather(chunk, b):
            pltpu.make_async_copy(
                t_hbm.at[ii_v.at[chunk]], tbuf.at[b], sems_t[b]).wait()
            pltpu.make_async_copy(
                q_hbm.at[jj_v.at[chunk]], qbuf.at[b], sems_q[b]).wait()

        def out_dst(chunk):
            return out_hbm.at[pl.ds(node_base + chunk * CHUNK, CHUNK)]

        issue_gather(0, 0)
        issue_gather(1, 1)

        def outer(o, _):
            for b in range(2):
                chunk = o * 2 + b
                wait_gather(chunk, b)

                @pl.when(o > 0)
                def _():
                    pltpu.make_async_copy(
                        obuf.at[b], out_dst(chunk - 2), sems_o[b]).wait()

                def node_body(m, _):
                    accs = [jnp.zeros((16,), jnp.float32) for _ in range(8)]
                    r0 = m * K
                    for k in range(K):
                        r = r0 + k
                        for j in range(2):
                            av, bv = unpack(tbuf[b, r, pl.ds(j * 32, 32)])
                            accs[2 * j] = accs[2 * j] + av
                            accs[2 * j + 1] = accs[2 * j + 1] + bv
                        for j in range(2):
                            ta, tb = unpack(tbuf[b, r, pl.ds(64 + j * 32, 32)])
                            qa, qb = unpack(qbuf[b, r, pl.ds(j * 32, 32)])
                            accs[4 + 2 * j] = accs[4 + 2 * j] + jnp.maximum(
                                qa - ta, 0.0)
                            accs[5 + 2 * j] = accs[5 + 2 * j] + jnp.maximum(
                                qb - tb, 0.0)
                    for ci in range(8):
                        obuf[b, m, pl.ds(ci * 16, 16)] = accs[ci] * inv_k
                    return 0

                lax.fori_loop(0, CHUNK, node_body, 0)
                pltpu.async_copy(obuf.at[b], out_dst(chunk), sems_o[b])

                @pl.when(chunk + 2 < n_chunks)
                def _():
                    issue_gather(chunk + 2, b)
            return 0

        lax.fori_loop(0, n_chunks // 2, outer, 0)
        for b in range(2):
            pltpu.make_async_copy(
                obuf.at[b], out_dst(n_chunks - 2 + b), sems_o[b]).wait()

    return sc_kernel


def kernel(x, edge_index, W, b):
    B, C, N, _ = x.shape
    K = edge_index.shape[-1]
    NP = ((N + NW * CHUNK - 1) // (NW * CHUNK)) * (NW * CHUNK)

    M = jnp.zeros((128, 128), jnp.float32)
    M = M.at[0:64, 0:32].set(W[0:32].T).at[64:128, 32:64].set(W[32:64].T)
    M = M.at[0:64, 64:96].set(W[64:96].T).at[64:128, 96:128].set(W[96:128].T)
    cfull = jnp.concatenate([b[:64], -b[64:]])
    M = M[:, _PERM]
    cvec = cfull[_PERM].reshape(1, 128)
    b2vec = b[_PERM[64:]].reshape(1, 64)

    xp = jnp.pad(x[0, :, :, 0], ((0, 0), (0, NP - N)))

    T, Q = _make_tables(xp, M, cvec, b2vec, NP, 512)

    NPW = NP // NW
    n_chunks = NPW // CHUNK
    ii = jnp.pad(edge_index[1, 0], ((0, NP - N), (0, 0)))
    jj = jnp.pad(edge_index[0, 0], ((0, NP - N), (0, 0)))
    ii = ii.reshape(NW, n_chunks, CHUNK * K)
    jj = jj.reshape(NW, n_chunks, CHUNK * K)

    out = _make_sc_kernel(NP, K)(T, Q, ii, jj)

    return out[:N].T[None, :, :, None]


# speedup vs baseline: 2.4531x; 1.8213x over previous
import functools

import jax
import jax.numpy as jnp
import numpy as np
from jax import lax
from jax.experimental import pallas as pl
from jax.experimental.pallas import tpu as pltpu
from jax.experimental.pallas import tpu_sc as plsc

NC = 2
NS = 16
NW = NC * NS
CHUNK = 8

_PERM = np.empty(128, np.int32)
for _j in range(4):
    for _t in range(16):
        _PERM[32 * _j + 2 * _t] = 32 * _j + _t
        _PERM[32 * _j + 2 * _t + 1] = 32 * _j + 16 + _t


def _tables_body(x_ref, m_ref, c_ref, b2_ref, t_ref, q_ref):
    y = lax.dot_general(x_ref[...], m_ref[...], (((0,), (0,)), ((), ())),
                        preferred_element_type=jnp.float32)
    y = y + c_ref[...]
    col = lax.broadcasted_iota(jnp.int32, y.shape, 1)
    t_ref[...] = jnp.where(col < 64, jnp.maximum(y, 0.0), y).astype(jnp.bfloat16)
    q_ref[...] = (y[:, 64:] + b2_ref[...]).astype(jnp.bfloat16)


def _make_tables(xp, M, cvec, b2vec, NP, N):
    return pl.pallas_call(
        _tables_body,
        grid=(1,),
        in_specs=[
            pl.BlockSpec((128, N), lambda i: (0, 0)),
            pl.BlockSpec((128, 128), lambda i: (0, 0)),
            pl.BlockSpec((1, 128), lambda i: (0, 0)),
            pl.BlockSpec((1, 64), lambda i: (0, 0)),
        ],
        out_specs=[
            pl.BlockSpec((N, 128), lambda i: (0, 0)),
            pl.BlockSpec((N, 64), lambda i: (0, 0)),
        ],
        out_shape=[
            jax.ShapeDtypeStruct((NP, 128), jnp.bfloat16),
            jax.ShapeDtypeStruct((NP, 64), jnp.bfloat16),
        ],
    )(xp, M, cvec, b2vec)


def _make_sc_kernel(NP, K):
    NPW = NP // NW
    n_chunks = NPW // CHUNK
    E = CHUNK * K

    mesh = plsc.VectorSubcoreMesh(core_axis_name="c", subcore_axis_name="s")

    @functools.partial(
        pl.kernel,
        out_type=jax.ShapeDtypeStruct((NP, 128), jnp.float32),
        mesh=mesh,
        scratch_types=[
            pltpu.VMEM((n_chunks, E), jnp.int32),
            pltpu.VMEM((n_chunks, E), jnp.int32),
            pltpu.VMEM((2, E, 128), jnp.bfloat16),
            pltpu.VMEM((2, E, 64), jnp.bfloat16),
            pltpu.VMEM((2, CHUNK, 128), jnp.float32),
            pltpu.SemaphoreType.DMA,
            pltpu.SemaphoreType.DMA,
            pltpu.SemaphoreType.DMA,
            pltpu.SemaphoreType.DMA,
            pltpu.SemaphoreType.DMA,
            pltpu.SemaphoreType.DMA,
        ],
        compiler_params=pltpu.CompilerParams(
            use_tc_tiling_on_sc=False, needs_layout_passes=False),
    )
    def sc_kernel(t_hbm, q_hbm, ij_hbm, out_hbm,
                  ii_v, jj_v, tbuf, qbuf, obuf,
                  sem_t0, sem_t1, sem_q0, sem_q1, sem_o0, sem_o1):
        sems_t = (sem_t0, sem_t1)
        sems_q = (sem_q0, sem_q1)
        sems_o = (sem_o0, sem_o1)
        wid = lax.axis_index("s") * NC + (1 - lax.axis_index("c"))
        pltpu.sync_copy(ij_hbm.at[1, wid], ii_v)
        pltpu.sync_copy(ij_hbm.at[0, wid], jj_v)
        node_base = wid * NPW
        inv_k = 1.0 / K
        unpack = functools.partial(
            plsc.unpack, format=plsc.PackFormat.INTERLEAVED)

        def issue_gather(chunk, b):
            pltpu.async_copy(t_hbm.at[ii_v.at[chunk]], tbuf.at[b], sems_t[b])
            pltpu.async_copy(q_hbm.at[jj_v.at[chunk]], qbuf.at[b], sems_q[b])

        def wait_gather(chunk, b):
            pltpu.make_async_copy(
                t_hbm.at[ii_v.at[chunk]], tbuf.at[b], sems_t[b]).wait()
            pltpu.make_async_copy(
                q_hbm.at[jj_v.at[chunk]], qbuf.at[b], sems_q[b]).wait()

        def out_dst(chunk):
            return out_hbm.at[pl.ds(node_base + chunk * CHUNK, CHUNK)]

        issue_gather(0, 0)
        issue_gather(1, 1)

        def outer(o, _):
            for b in range(2):
                chunk = o * 2 + b
                wait_gather(chunk, b)

                @pl.when(o > 0)
                def _():
                    pltpu.make_async_copy(
                        obuf.at[b], out_dst(chunk - 2), sems_o[b]).wait()

                def node_body(m, _):
                    accs = [jnp.zeros((16,), jnp.float32) for _ in range(8)]
                    r0 = m * K
                    for k in range(K):
                        r = r0 + k
                        for j in range(2):
                            av, bv = unpack(tbuf[b, r, pl.ds(j * 32, 32)])
                            accs[2 * j] = accs[2 * j] + av
                            accs[2 * j + 1] = accs[2 * j + 1] + bv
                        for j in range(2):
                            ta, tb = unpack(tbuf[b, r, pl.ds(64 + j * 32, 32)])
                            qa, qb = unpack(qbuf[b, r, pl.ds(j * 32, 32)])
                            accs[4 + 2 * j] = accs[4 + 2 * j] + jnp.maximum(
                                qa - ta, 0.0)
                            accs[5 + 2 * j] = accs[5 + 2 * j] + jnp.maximum(
                                qb - tb, 0.0)
                    for ci in range(8):
                        obuf[b, m, pl.ds(ci * 16, 16)] = accs[ci] * inv_k
                    return 0

                lax.fori_loop(0, CHUNK, node_body, 0)
                pltpu.async_copy(obuf.at[b], out_dst(chunk), sems_o[b])

                @pl.when(chunk + 2 < n_chunks)
                def _():
                    issue_gather(chunk + 2, b)
            return 0

        lax.fori_loop(0, n_chunks // 2, outer, 0)
        for b in range(2):
            pltpu.make_async_copy(
                obuf.at[b], out_dst(n_chunks - 2 + b), sems_o[b]).wait()

    return sc_kernel


def kernel(x, edge_index, W, b):
    B, C, N, _ = x.shape
    K = edge_index.shape[-1]
    NP = ((N + NW * CHUNK - 1) // (NW * CHUNK)) * (NW * CHUNK)

    M = jnp.zeros((128, 128), jnp.float32)
    M = M.at[0:64, 0:32].set(W[0:32].T).at[64:128, 32:64].set(W[32:64].T)
    M = M.at[0:64, 64:96].set(W[64:96].T).at[64:128, 96:128].set(W[96:128].T)
    cfull = jnp.concatenate([b[:64], -b[64:]])
    M = M[:, _PERM]
    cvec = cfull[_PERM].reshape(1, 128)
    b2vec = b[_PERM[64:]].reshape(1, 64)

    xb = x[0, :, :, 0]

    T, Q = _make_tables(xb, M, cvec, b2vec, NP, N)

    NPW = NP // NW
    n_chunks = NPW // CHUNK
    ij = jnp.pad(edge_index[:, 0], ((0, 0), (0, NP - N), (0, 0)), mode='wrap')
    ij = ij.reshape(2, NW, n_chunks, CHUNK * K)

    out = _make_sc_kernel(NP, K)(T, Q, ij)

    return out[:N].T[None, :, :, None]


# speedup vs baseline: 2.4882x; 1.0143x over previous
import functools

import jax
import jax.numpy as jnp
import numpy as np
from jax import lax
from jax.experimental import pallas as pl
from jax.experimental.pallas import tpu as pltpu
from jax.experimental.pallas import tpu_sc as plsc

NC = 2
NS = 16
NW = NC * NS
CHUNK = 8

_PERM = np.empty(128, np.int32)
for _j in range(4):
    for _t in range(16):
        _PERM[32 * _j + 2 * _t] = 32 * _j + _t
        _PERM[32 * _j + 2 * _t + 1] = 32 * _j + 16 + _t


def _tables_body(x_ref, m_ref, c_ref, b2_ref, t_ref, q_ref):
    N = x_ref.shape[1]
    y = lax.dot_general(x_ref[...], m_ref[...], (((0,), (0,)), ((), ())),
                        preferred_element_type=jnp.float32)
    y = y + c_ref[...]
    col = lax.broadcasted_iota(jnp.int32, y.shape, 1)
    t = jnp.where(col < 64, jnp.maximum(y, 0.0), y).astype(jnp.bfloat16)
    t_ref[...] = t.reshape(N // 16, 16, 128)
    q = (y[:, 64:] + b2_ref[...]).astype(jnp.bfloat16)
    q_ref[...] = jnp.concatenate([q, q], axis=1).reshape(N // 16, 16, 128)


def _make_tables(xp, M, cvec, b2vec, NP, N):
    return pl.pallas_call(
        _tables_body,
        grid=(1,),
        in_specs=[
            pl.BlockSpec((128, N), lambda i: (0, 0)),
            pl.BlockSpec((128, 128), lambda i: (0, 0)),
            pl.BlockSpec((1, 128), lambda i: (0, 0)),
            pl.BlockSpec((1, 64), lambda i: (0, 0)),
        ],
        out_specs=[
            pl.BlockSpec((N // 16, 16, 128), lambda i: (0, 0, 0)),
            pl.BlockSpec((N // 16, 16, 128), lambda i: (0, 0, 0)),
        ],
        out_shape=[
            jax.ShapeDtypeStruct((NP // 16, 16, 128), jnp.bfloat16),
            jax.ShapeDtypeStruct((NP // 16, 16, 128), jnp.bfloat16),
        ],
    )(xp, M, cvec, b2vec)


def _make_sc_kernel(NP, K):
    NPW = NP // NW
    n_chunks = NPW // CHUNK
    E = CHUNK * K

    mesh = plsc.VectorSubcoreMesh(core_axis_name="c", subcore_axis_name="s")

    @functools.partial(
        pl.kernel,
        out_type=jax.ShapeDtypeStruct((NP, 128), jnp.float32),
        mesh=mesh,
        scratch_types=[
            pltpu.VMEM((n_chunks, E), jnp.int32),
            pltpu.VMEM((n_chunks, E), jnp.int32),
            pltpu.VMEM((2, E, 128), jnp.bfloat16),
            pltpu.VMEM((2, E, 64), jnp.bfloat16),
            pltpu.VMEM((2, CHUNK, 128), jnp.float32),
            pltpu.SemaphoreType.DMA,
            pltpu.SemaphoreType.DMA,
            pltpu.SemaphoreType.DMA,
            pltpu.SemaphoreType.DMA,
            pltpu.SemaphoreType.DMA,
            pltpu.SemaphoreType.DMA,
        ],
        compiler_params=pltpu.CompilerParams(
            use_tc_tiling_on_sc=False, needs_layout_passes=False),
    )
    def sc_kernel(t_hbm, q_hbm, ij_hbm, out_hbm,
                  ii_v, jj_v, tbuf, qbuf, obuf,
                  sem_t0, sem_t1, sem_q0, sem_q1, sem_o0, sem_o1):
        sems_t = (sem_t0, sem_t1)
        sems_q = (sem_q0, sem_q1)
        sems_o = (sem_o0, sem_o1)
        wid = lax.axis_index("s") * NC + (1 - lax.axis_index("c"))
        pltpu.sync_copy(ij_hbm.at[1, wid], ii_v)
        pltpu.sync_copy(ij_hbm.at[0, wid], jj_v)
        node_base = wid * NPW
        inv_k = 1.0 / K
        unpack = functools.partial(
            plsc.unpack, format=plsc.PackFormat.INTERLEAVED)

        def issue_gather(chunk, b):
            pltpu.async_copy(t_hbm.at[ii_v.at[chunk]], tbuf.at[b], sems_t[b])
            pltpu.async_copy(q_hbm.at[jj_v.at[chunk]], qbuf.at[b], sems_q[b])

        def wait_gather(chunk, b):
            pltpu.make_async_copy(
                t_hbm.at[ii_v.at[chunk]], tbuf.at[b], sems_t[b]).wait()
            pltpu.make_async_copy(
                q_hbm.at[jj_v.at[chunk]], qbuf.at[b], sems_q[b]).wait()

        def out_dst(chunk):
            return out_hbm.at[pl.ds(node_base + chunk * CHUNK, CHUNK)]

        issue_gather(0, 0)
        issue_gather(1, 1)

        def outer(o, _):
            for b in range(2):
                chunk = o * 2 + b
                wait_gather(chunk, b)

                @pl.when(o > 0)
                def _():
                    pltpu.make_async_copy(
                        obuf.at[b], out_dst(chunk - 2), sems_o[b]).wait()

                def node_body(m, _):
                    accs = [jnp.zeros((16,), jnp.float32) for _ in range(8)]
                    r0 = m * K
                    for k in range(K):
                        r = r0 + k
                        for j in range(2):
                            av, bv = unpack(tbuf[b, r, pl.ds(j * 32, 32)])
                            accs[2 * j] = accs[2 * j] + av
                            accs[2 * j + 1] = accs[2 * j + 1] + bv
                        for j in range(2):
                            ta, tb = unpack(tbuf[b, r, pl.ds(64 + j * 32, 32)])
                            qa, qb = unpack(qbuf[b, r, pl.ds(j * 32, 32)])
                            accs[4 + 2 * j] = accs[4 + 2 * j] + jnp.maximum(
                                qa - ta, 0.0)
                            accs[5 + 2 * j] = accs[5 + 2 * j] + jnp.maximum(
                                qb - tb, 0.0)
                    for ci in range(8):
                        obuf[b, m, pl.ds(ci * 16, 16)] = accs[ci] * inv_k
                    return 0

                lax.fori_loop(0, CHUNK, node_body, 0)
                pltpu.async_copy(obuf.at[b], out_dst(chunk), sems_o[b])

                @pl.when(chunk + 2 < n_chunks)
                def _():
                    issue_gather(chunk + 2, b)
            return 0

        lax.fori_loop(0, n_chunks // 2, outer, 0)
        for b in range(2):
            pltpu.make_async_copy(
                obuf.at[b], out_dst(n_chunks - 2 + b), sems_o[b]).wait()

    return sc_kernel


def kernel(x, edge_index, W, b):
    B, C, N, _ = x.shape
    K = edge_index.shape[-1]
    NP = ((N + NW * CHUNK - 1) // (NW * CHUNK)) * (NW * CHUNK)

    rb = _PERM // 32 % 2
    dmask = (np.arange(128)[:, None] // 64 == rb[None, :]).astype(np.float32)
    Wp = W[_PERM]
    M = jnp.tile(Wp.T, (2, 1)) * dmask
    cfull = jnp.where(np.arange(128) < 64, b, -b)
    cvec = cfull[_PERM].reshape(1, 128)
    b2vec = b[_PERM[64:]].reshape(1, 64)

    xb = x[0, :, :, 0]

    T3, Q3 = _make_tables(xb, M, cvec, b2vec, NP, N)
    T = T3.reshape(NP, 128)
    Q = Q3.reshape(2 * NP, 64)

    NPW = NP // NW
    n_chunks = NPW // CHUNK
    ij = jnp.pad(edge_index[:, 0], ((0, 0), (0, NP - N), (0, 0)), mode='wrap')
    ij = ij * jnp.array([2, 1], jnp.int32).reshape(2, 1, 1)
    ij = ij.reshape(2, NW, n_chunks, CHUNK * K)

    out = _make_sc_kernel(NP, K)(T, Q, ij)

    return out[:N].T[None, :, :, None]


# speedup vs baseline: 3.0696x; 1.2336x over previous
import functools

import jax
import jax.numpy as jnp
import numpy as np
from jax import lax
from jax.experimental import pallas as pl
from jax.experimental.pallas import tpu as pltpu
from jax.experimental.pallas import tpu_sc as plsc

NC = 2
NS = 16
NW = NC * NS
CHUNK = 8

_COLS = np.empty(128, np.int32)
for _c in range(64):
    _COLS[_c] = 32 * (_c // 16) + _c % 16
    _COLS[64 + _c] = 32 * (_c // 16) + 16 + _c % 16


def _rne_pack(a, b):
    ua = lax.bitcast_convert_type(a, jnp.int32)
    ub = lax.bitcast_convert_type(b, jnp.int32)
    ra = ua + jnp.int32(0x7FFF) + ((ua >> 16) & jnp.int32(1))
    rb = ub + jnp.int32(0x7FFF) + ((ub >> 16) & jnp.int32(1))
    lo = (ra >> 16) & jnp.int32(0xFFFF)
    hi = rb & jnp.int32(-65536)
    return lo | hi


def _tables_body(x_ref, m_ref, c_ref, b2q_ref, t_ref, q_ref):
    N = x_ref.shape[1]
    y = lax.dot_general(x_ref[...], m_ref[...], (((0,), (0,)), ((), ())),
                        preferred_element_type=jnp.float32)
    y = y + c_ref[...]
    col = lax.broadcasted_iota(jnp.int32, y.shape, 1)
    t = jnp.where(col % 64 < 32, jnp.maximum(y, 0.0), y)
    A, B = t[:, :64], t[:, 64:]
    TW = _rne_pack(A, B)
    t_ref[...] = jnp.concatenate([TW[: N // 2], TW[N // 2:]], axis=1)
    Qa = A[:, 32:] + b2q_ref[:, :32]
    Qb = B[:, 32:] + b2q_ref[:, 32:]
    QW = _rne_pack(Qa, Qb)
    NQ = N // 4
    q_ref[...] = jnp.concatenate(
        [QW[:NQ], QW[NQ:2 * NQ], QW[2 * NQ:3 * NQ], QW[3 * NQ:]], axis=1)


def _make_tables(xp, M, cvec, b2q, N):
    return pl.pallas_call(
        _tables_body,
        grid=(1,),
        in_specs=[
            pl.BlockSpec((128, N), lambda i: (0, 0)),
            pl.BlockSpec((128, 128), lambda i: (0, 0)),
            pl.BlockSpec((1, 128), lambda i: (0, 0)),
            pl.BlockSpec((1, 64), lambda i: (0, 0)),
        ],
        out_specs=[
            pl.BlockSpec((N // 2, 128), lambda i: (0, 0)),
            pl.BlockSpec((N // 4, 128), lambda i: (0, 0)),
        ],
        out_shape=[
            jax.ShapeDtypeStruct((N // 2, 128), jnp.int32),
            jax.ShapeDtypeStruct((N // 4, 128), jnp.int32),
        ],
    )(xp, M, cvec, b2q)


def _make_sc_kernel(NP, N, K):
    NPW = NP // NW
    n_chunks = NPW // CHUNK
    E = CHUNK * K

    mesh = plsc.VectorSubcoreMesh(core_axis_name="c", subcore_axis_name="s")

    @functools.partial(
        pl.kernel,
        out_type=jax.ShapeDtypeStruct((NP, 128), jnp.float32),
        mesh=mesh,
        scratch_types=[
            pltpu.VMEM((n_chunks, E), jnp.int32),
            pltpu.VMEM((n_chunks, E), jnp.int32),
            pltpu.VMEM((2, E, 64), jnp.int32),
            pltpu.VMEM((2, E, 32), jnp.int32),
            pltpu.VMEM((2, CHUNK, 128), jnp.float32),
            pltpu.SemaphoreType.DMA,
            pltpu.SemaphoreType.DMA,
            pltpu.SemaphoreType.DMA,
            pltpu.SemaphoreType.DMA,
            pltpu.SemaphoreType.DMA,
            pltpu.SemaphoreType.DMA,
        ],
        compiler_params=pltpu.CompilerParams(
            use_tc_tiling_on_sc=False, needs_layout_passes=False),
    )
    def sc_kernel(t_hbm, q_hbm, ij_hbm, out_hbm,
                  ii_v, jj_v, tbuf, qbuf, obuf,
                  sem_t0, sem_t1, sem_q0, sem_q1, sem_o0, sem_o1):
        sems_t = (sem_t0, sem_t1)
        sems_q = (sem_q0, sem_q1)
        sems_o = (sem_o0, sem_o1)
        wid = lax.axis_index("s") * NC + (1 - lax.axis_index("c"))
        pltpu.sync_copy(ij_hbm.at[1, wid], ii_v)
        pltpu.sync_copy(ij_hbm.at[0, wid], jj_v)
        node_base = wid * NPW
        inv_k = 1.0 / K
        unpack = functools.partial(
            plsc.unpack, format=plsc.PackFormat.INTERLEAVED)

        def issue_gather(chunk, b):
            pltpu.async_copy(t_hbm.at[ii_v.at[chunk]], tbuf.at[b], sems_t[b])
            pltpu.async_copy(q_hbm.at[jj_v.at[chunk]], qbuf.at[b], sems_q[b])

        def wait_gather(chunk, b):
            pltpu.make_async_copy(
                t_hbm.at[ii_v.at[chunk]], tbuf.at[b], sems_t[b]).wait()
            pltpu.make_async_copy(
                q_hbm.at[jj_v.at[chunk]], qbuf.at[b], sems_q[b]).wait()

        def out_dst(chunk):
            return out_hbm.at[pl.ds(node_base + chunk * CHUNK, CHUNK)]

        issue_gather(0, 0)
        issue_gather(1, 1)

        def outer(o, _):
            for b in range(2):
                chunk = o * 2 + b
                wait_gather(chunk, b)

                @pl.when(o > 0)
                def _():
                    pltpu.make_async_copy(
                        obuf.at[b], out_dst(chunk - 2), sems_o[b]).wait()

                def node_body(m, _):
                    accs = [jnp.zeros((16,), jnp.float32) for _ in range(8)]
                    r0 = m * K
                    for k in range(K):
                        r = r0 + k
                        for u in range(2):
                            w = plsc.bitcast(
                                tbuf[b, r, pl.ds(u * 16, 16)], jnp.bfloat16)
                            av, bv = unpack(w)
                            accs[2 * u] = accs[2 * u] + av
                            accs[2 * u + 1] = accs[2 * u + 1] + bv
                        for u in range(2):
                            tw = plsc.bitcast(
                                tbuf[b, r, pl.ds(32 + u * 16, 16)],
                                jnp.bfloat16)
                            ta, tb = unpack(tw)
                            qw = plsc.bitcast(
                                qbuf[b, r, pl.ds(u * 16, 16)], jnp.bfloat16)
                            qa, qb = unpack(qw)
                            accs[4 + 2 * u] = accs[4 + 2 * u] + jnp.maximum(
                                qa - ta, 0.0)
                            accs[5 + 2 * u] = accs[5 + 2 * u] + jnp.maximum(
                                qb - tb, 0.0)
                    for ci in range(8):
                        obuf[b, m, pl.ds(ci * 16, 16)] = accs[ci] * inv_k
                    return 0

                lax.fori_loop(0, CHUNK, node_body, 0)
                pltpu.async_copy(obuf.at[b], out_dst(chunk), sems_o[b])

                @pl.when(chunk + 2 < n_chunks)
                def _():
                    issue_gather(chunk + 2, b)
            return 0

        lax.fori_loop(0, n_chunks // 2, outer, 0)
        for b in range(2):
            pltpu.make_async_copy(
                obuf.at[b], out_dst(n_chunks - 2 + b), sems_o[b]).wait()

    return sc_kernel


def kernel(x, edge_index, W, b):
    B, C, N, _ = x.shape
    K = edge_index.shape[-1]
    NP = ((N + NW * CHUNK - 1) // (NW * CHUNK)) * (NW * CHUNK)

    rb = _COLS // 32 % 2
    dmask = (np.arange(128)[:, None] // 64 == rb[None, :]).astype(np.float32)
    Wp = W[_COLS]
    M = jnp.tile(Wp.T, (2, 1)) * dmask
    chan = _COLS
    cvec = (jnp.where(chan < 64, b[chan], -b[chan])).reshape(1, 128)
    b2q = jnp.concatenate([b[_COLS[32:64]], b[_COLS[96:128]]]).reshape(1, 64)

    xb = x[0, :, :, 0]

    T2, Q2 = _make_tables(xb, M, cvec, b2q, N)
    T = T2.reshape(N, 64)
    Q = Q2.reshape(N, 32)

    HN, QN = N // 2, N // 4
    e_j, e_i = edge_index[0, 0], edge_index[1, 0]
    ti = 2 * (e_i % HN) + e_i // HN
    tj = 4 * (e_j % QN) + e_j // QN
    ij = jnp.stack([tj, ti])
    ij = jnp.pad(ij, ((0, 0), (0, NP - N), (0, 0)), mode='wrap')
    NPW = NP // NW
    n_chunks = NPW // CHUNK
    ij = ij.reshape(2, NW, n_chunks, CHUNK * K)

    out = _make_sc_kernel(NP, N, K)(T, Q, ij)

    return out[:N].T[None, :, :, None]
